# Initial kernel scaffold; baseline (speedup 1.0000x reference)
import jax
import jax.numpy as jnp
from jax.experimental import pallas as pl


def kernel(char_sequences, char_emb_table):
    raise NotImplementedError("write your pallas kernel here")


# speedup vs baseline: 2.8604x; 2.8604x over previous
import functools

import jax
import jax.numpy as jnp
from jax import lax
from jax.experimental import pallas as pl
from jax.experimental.pallas import tpu as pltpu
from jax.experimental.pallas import tpu_sc as plsc

CHUNK = 128


def _make_lookup(tot, vocab, dim, num_workers):
    assert tot % (num_workers * CHUNK) == 0
    rows_per_worker = tot // num_workers
    chunks_per_worker = rows_per_worker // CHUNK

    mesh = plsc.VectorSubcoreMesh(core_axis_name="c", subcore_axis_name="s")

    @functools.partial(
        pl.kernel,
        mesh=mesh,
        out_type=jax.ShapeDtypeStruct((tot, dim), jnp.float32),
        scratch_types=[
            pltpu.VMEM((chunks_per_worker, CHUNK), jnp.int32),
            pltpu.VMEM((CHUNK, dim), jnp.float32),
            pltpu.SemaphoreType.DMA,
        ],
    )
    def lookup(idx_hbm, table_hbm, out_hbm, idx_v, buf, gsem):
        nc = lax.axis_size("c")
        wid = lax.axis_index("s") * nc + lax.axis_index("c")
        pltpu.sync_copy(idx_hbm.at[pl.ds(wid * chunks_per_worker, chunks_per_worker)], idx_v)
        row_base = wid * rows_per_worker

        def step(j, carry):
            pltpu.async_copy(table_hbm.at[idx_v.at[j]], buf, gsem).wait()
            pltpu.sync_copy(buf, out_hbm.at[pl.ds(row_base + j * CHUNK, CHUNK)])
            return carry

        lax.fori_loop(0, chunks_per_worker, step, 0)

    return lookup


def kernel(char_sequences, char_emb_table):
    batch, word_len = char_sequences.shape
    vocab, dim = char_emb_table.shape
    tot = batch * word_len
    idx2d = char_sequences.reshape(tot // CHUNK, CHUNK)
    info = plsc.get_sparse_core_info()
    num_workers = info.num_cores * info.num_subcores
    out = _make_lookup(tot, vocab, dim, num_workers)(idx2d, char_emb_table)
    return out.reshape(batch, word_len, dim)


# speedup vs baseline: 2.9803x; 1.0419x over previous
import functools

import jax
import jax.numpy as jnp
from jax import lax
from jax.experimental import pallas as pl
from jax.experimental.pallas import tpu as pltpu
from jax.experimental.pallas import tpu_sc as plsc

CHUNK = 128
NBUF = 4


def _make_lookup(tot, vocab, dim, num_workers):
    assert tot % (num_workers * CHUNK * NBUF) == 0
    rows_per_worker = tot // num_workers
    chunks = rows_per_worker // CHUNK
    rounds = chunks // NBUF

    mesh = plsc.VectorSubcoreMesh(core_axis_name="c", subcore_axis_name="s")

    @functools.partial(
        pl.kernel,
        mesh=mesh,
        out_type=jax.ShapeDtypeStruct((tot, dim), jnp.float32),
        scratch_types=(
            [pltpu.VMEM((chunks, CHUNK), jnp.int32)]
            + [pltpu.VMEM((CHUNK, dim), jnp.float32) for _ in range(NBUF)]
            + [pltpu.SemaphoreType.DMA for _ in range(2 * NBUF)]
        ),
    )
    def lookup(idx_hbm, table_hbm, out_hbm, idx_v, *scratch):
        bufs = scratch[:NBUF]
        gsems = scratch[NBUF : 2 * NBUF]
        ssems = scratch[2 * NBUF :]
        nc = lax.axis_size("c")
        wid = lax.axis_index("s") * nc + lax.axis_index("c")
        pltpu.sync_copy(idx_hbm.at[pl.ds(wid * chunks, chunks)], idx_v)
        row_base = wid * rows_per_worker

        def gather(b, j):
            pltpu.async_copy(table_hbm.at[idx_v.at[j]], bufs[b], gsems[b])

        def out_slice(j):
            return out_hbm.at[pl.ds(row_base + j * CHUNK, CHUNK)]

        for b in range(NBUF):
            gather(b, b)

        def round_step(r, carry):
            for b in range(NBUF):
                j = r * NBUF + b
                pltpu.make_async_copy(table_hbm.at[idx_v.at[j]], bufs[b], gsems[b]).wait()
                pltpu.async_copy(bufs[b], out_slice(j), ssems[b])

                @pl.when(j + NBUF < chunks)
                def _():
                    pltpu.make_async_copy(bufs[b], out_slice(j), ssems[b]).wait()
                    gather(b, j + NBUF)

            return carry

        lax.fori_loop(0, rounds, round_step, 0)
        for b in range(NBUF):
            pltpu.make_async_copy(bufs[b], out_slice(chunks - NBUF + b), ssems[b]).wait()

    return lookup


def kernel(char_sequences, char_emb_table):
    batch, word_len = char_sequences.shape
    vocab, dim = char_emb_table.shape
    tot = batch * word_len
    idx2d = char_sequences.reshape(tot // CHUNK, CHUNK)
    info = plsc.get_sparse_core_info()
    num_workers = info.num_cores * info.num_subcores
    out = _make_lookup(tot, vocab, dim, num_workers)(idx2d, char_emb_table)
    return out.reshape(batch, word_len, dim)


# speedup vs baseline: 4.6914x; 1.5742x over previous
import functools

import jax
import jax.numpy as jnp
from jax import lax
from jax.experimental import pallas as pl
from jax.experimental.pallas import tpu as pltpu
from jax.experimental.pallas import tpu_sc as plsc

NBUF = 4
WPC = 4


def _make_lookup(batch, word_len, vocab, dim, num_workers):
    chunk = WPC * word_len
    assert chunk <= 128
    assert batch % (num_workers * WPC) == 0
    words_per_worker = batch // num_workers
    chunks = words_per_worker // WPC
    rounds = chunks // NBUF
    assert chunks % NBUF == 0

    mesh = plsc.VectorSubcoreMesh(core_axis_name="c", subcore_axis_name="s")

    @functools.partial(
        pl.kernel,
        mesh=mesh,
        out_type=jax.ShapeDtypeStruct((batch, word_len, dim), jnp.float32),
        scratch_types=(
            [pltpu.VMEM((chunks, chunk), jnp.int32)]
            + [pltpu.VMEM((chunk, dim), jnp.float32) for _ in range(NBUF)]
            + [pltpu.SemaphoreType.DMA for _ in range(2 * NBUF)]
        ),
    )
    def lookup(idx_hbm, table_hbm, out_hbm, idx_v, *scratch):
        bufs = scratch[:NBUF]
        gsems = scratch[NBUF : 2 * NBUF]
        ssems = scratch[2 * NBUF :]
        nc = lax.axis_size("c")
        wid = lax.axis_index("s") * nc + lax.axis_index("c")
        pltpu.sync_copy(idx_hbm.at[pl.ds(wid * chunks, chunks)], idx_v)
        word_base = wid * words_per_worker

        def gather(b, j):
            pltpu.async_copy(table_hbm.at[idx_v.at[j]], bufs[b], gsems[b])

        def stores(b, j):
            for w in range(WPC):
                pltpu.async_copy(
                    bufs[b].at[pl.ds(w * word_len, word_len)],
                    out_hbm.at[word_base + j * WPC + w],
                    ssems[b],
                )

        def wait_stores(b):
            for w in range(WPC):
                pltpu.make_async_copy(
                    bufs[b].at[pl.ds(w * word_len, word_len)],
                    out_hbm.at[0],
                    ssems[b],
                ).wait()

        for b in range(NBUF):
            gather(b, b)

        def round_step(r, carry):
            for b in range(NBUF):
                j = r * NBUF + b
                pltpu.make_async_copy(table_hbm.at[idx_v.at[j]], bufs[b], gsems[b]).wait()
                stores(b, j)

                @pl.when(j + NBUF < chunks)
                def _():
                    wait_stores(b)
                    gather(b, j + NBUF)

            return carry

        lax.fori_loop(0, rounds, round_step, 0)
        for b in range(NBUF):
            wait_stores(b)

    return lookup


def kernel(char_sequences, char_emb_table):
    batch, word_len = char_sequences.shape
    vocab, dim = char_emb_table.shape
    idx2d = char_sequences.reshape(batch // WPC, WPC * word_len)
    info = plsc.get_sparse_core_info()
    num_workers = info.num_cores * info.num_subcores
    return _make_lookup(batch, word_len, vocab, dim, num_workers)(idx2d, char_emb_table)
